# baseline (device time: 10268 ns/iter reference)
import jax
import jax.numpy as jnp
from jax import lax
from jax.experimental import pallas as pl
from jax.experimental.pallas import tpu as pltpu

N_DEV = 4
BLOCK_M = 512
N_HALVES = 2


def kernel(x):
    m_per, n = x.shape
    n_half = n // N_HALVES
    n_row_blocks = m_per // BLOCK_M

    def body(x_ref, out_ref, partial_ref, recv_ref, send_sems, recv_sems):
        j = pl.program_id(0)
        i = pl.program_id(1)
        my_pos = lax.axis_index("i")
        barrier_sem = pltpu.get_barrier_semaphore()

        def make_rdma(half, d):
            slot = half * (N_DEV - 1) + (d - 1)
            return pltpu.make_async_remote_copy(
                src_ref=partial_ref.at[pl.ds(half, 1)],
                dst_ref=recv_ref.at[pl.ds(slot, 1)],
                send_sem=send_sems.at[slot],
                recv_sem=recv_sems.at[slot],
                device_id=((my_pos + d) % N_DEV,),
                device_id_type=pl.DeviceIdType.MESH,
            )

        @pl.when((j == 0) & (i == 0))
        def _():
            for d in range(1, N_DEV):
                pl.semaphore_signal(
                    barrier_sem, inc=1,
                    device_id=((my_pos + d) % N_DEV,),
                    device_id_type=pl.DeviceIdType.MESH,
                )

        blk = jnp.sum(x_ref[...], axis=0, keepdims=True)

        @pl.when(i == 0)
        def _():
            partial_ref[pl.ds(j, 1), :] = blk

        @pl.when(i != 0)
        def _():
            partial_ref[pl.ds(j, 1), :] += blk

        @pl.when((j == 0) & (i == n_row_blocks - 1))
        def _():
            pl.semaphore_wait(barrier_sem, N_DEV - 1)
            for d in range(1, N_DEV):
                make_rdma(0, d).start()

        @pl.when((j == N_HALVES - 1) & (i == n_row_blocks - 1))
        def _():
            for d in range(1, N_DEV):
                make_rdma(1, d).start()
            for half in range(N_HALVES):
                for d in range(1, N_DEV):
                    make_rdma(half, d).wait()

            for half in range(N_HALVES):
                lo = half * (N_DEV - 1)
                out_ref[0:1, half * n_half:(half + 1) * n_half] = (
                    partial_ref[half:half + 1, :]
                    + jnp.sum(recv_ref[lo:lo + 3, :], axis=0, keepdims=True)
                )

    return pl.pallas_call(
        body,
        grid=(N_HALVES, n_row_blocks),
        out_shape=jax.ShapeDtypeStruct((1, n), x.dtype),
        in_specs=[
            pl.BlockSpec(
                (BLOCK_M, n_half), lambda j, i: (i, j), memory_space=pltpu.VMEM
            )
        ],
        out_specs=pl.BlockSpec(
            (1, n), lambda j, i: (0, 0), memory_space=pltpu.VMEM
        ),
        scratch_shapes=[
            pltpu.VMEM((N_HALVES, n_half), x.dtype),
            pltpu.VMEM((N_HALVES * (N_DEV - 1), n_half), x.dtype),
            pltpu.SemaphoreType.DMA((N_HALVES * (N_DEV - 1),)),
            pltpu.SemaphoreType.DMA((N_HALVES * (N_DEV - 1),)),
        ],
        compiler_params=pltpu.CompilerParams(
            collective_id=0,
            dimension_semantics=("arbitrary", "arbitrary"),
        ),
    )(x)


# device time: 9481 ns/iter; 1.0830x vs baseline; 1.0830x over previous
import jax
import jax.numpy as jnp
from jax import lax
from jax.experimental import pallas as pl
from jax.experimental.pallas import tpu as pltpu

N_DEV = 4
BLOCK_M = 256


def kernel(x):
    m_per, n = x.shape
    n_blocks = m_per // BLOCK_M
    mid = n_blocks // 2

    def body(x_ref, out_ref, partial_ref, recv_ref, send_sems, recv_sems):
        i = pl.program_id(0)
        my_pos = lax.axis_index("i")
        barrier_sem = pltpu.get_barrier_semaphore()

        def make_rdma(half, d):
            slot = half * (N_DEV - 1) + (d - 1)
            return pltpu.make_async_remote_copy(
                src_ref=partial_ref.at[pl.ds(half, 1)],
                dst_ref=recv_ref.at[pl.ds(slot, 1)],
                send_sem=send_sems.at[slot],
                recv_sem=recv_sems.at[slot],
                device_id=((my_pos + d) % N_DEV,),
                device_id_type=pl.DeviceIdType.MESH,
            )

        @pl.when(i == 0)
        def _():
            for d in range(1, N_DEV):
                pl.semaphore_signal(
                    barrier_sem, inc=1,
                    device_id=((my_pos + d) % N_DEV,),
                    device_id_type=pl.DeviceIdType.MESH,
                )

        blk = jnp.sum(x_ref[...], axis=0, keepdims=True)
        half = jnp.where(i < mid, 0, 1)

        @pl.when((i == 0) | (i == mid))
        def _():
            partial_ref[pl.ds(half, 1), :] = blk

        @pl.when((i != 0) & (i != mid))
        def _():
            partial_ref[pl.ds(half, 1), :] += blk

        @pl.when(i == mid - 1)
        def _():
            pl.semaphore_wait(barrier_sem, N_DEV - 1)
            for d in range(1, N_DEV):
                make_rdma(0, d).start()

        @pl.when(i == n_blocks - 1)
        def _():
            for d in range(1, N_DEV):
                make_rdma(1, d).start()
            for h in range(2):
                for d in range(1, N_DEV):
                    make_rdma(h, d).wait()

            out_ref[...] = (
                partial_ref[0:1, :]
                + partial_ref[1:2, :]
                + jnp.sum(recv_ref[...], axis=0, keepdims=True)
            )

    return pl.pallas_call(
        body,
        grid=(n_blocks,),
        out_shape=jax.ShapeDtypeStruct((1, n), x.dtype),
        in_specs=[
            pl.BlockSpec((BLOCK_M, n), lambda i: (i, 0), memory_space=pltpu.VMEM)
        ],
        out_specs=pl.BlockSpec((1, n), lambda i: (0, 0), memory_space=pltpu.VMEM),
        scratch_shapes=[
            pltpu.VMEM((2, n), x.dtype),
            pltpu.VMEM((2 * (N_DEV - 1), n), x.dtype),
            pltpu.SemaphoreType.DMA((2 * (N_DEV - 1),)),
            pltpu.SemaphoreType.DMA((2 * (N_DEV - 1),)),
        ],
        compiler_params=pltpu.CompilerParams(
            collective_id=0,
            dimension_semantics=("arbitrary",),
        ),
    )(x)


# device time: 9450 ns/iter; 1.0866x vs baseline; 1.0033x over previous
import jax
import jax.numpy as jnp
from jax import lax
from jax.experimental import pallas as pl
from jax.experimental.pallas import tpu as pltpu

N_DEV = 4
BLOCK_M = 256


def kernel(x):
    m_per, n = x.shape
    n_blocks = m_per // BLOCK_M
    mid = n_blocks // 2

    def body(x_ref, out_ref, partial_ref, recv_ref, send_sems, recv_sems):
        i = pl.program_id(0)
        my_pos = lax.axis_index("i")
        barrier_sem = pltpu.get_barrier_semaphore()

        def make_rdma(half, d):
            slot = half * (N_DEV - 1) + (d - 1)
            return pltpu.make_async_remote_copy(
                src_ref=partial_ref.at[pl.ds(half, 1)],
                dst_ref=recv_ref.at[pl.ds(slot, 1)],
                send_sem=send_sems.at[slot],
                recv_sem=recv_sems.at[slot],
                device_id=((my_pos + d) % N_DEV,),
                device_id_type=pl.DeviceIdType.MESH,
            )

        @pl.when(i == 0)
        def _():
            for d in range(1, N_DEV):
                pl.semaphore_signal(
                    barrier_sem, inc=1,
                    device_id=((my_pos + d) % N_DEV,),
                    device_id_type=pl.DeviceIdType.MESH,
                )

        blk = jnp.sum(x_ref[...], axis=0, keepdims=True)
        half = jnp.where(i < mid, 0, 1)

        @pl.when((i == 0) | (i == mid))
        def _():
            partial_ref[pl.ds(half, 1), :] = blk

        @pl.when((i != 0) & (i != mid))
        def _():
            partial_ref[pl.ds(half, 1), :] += blk

        @pl.when(i == mid - 1)
        def _():
            pl.semaphore_wait(barrier_sem, N_DEV - 1)
            for d in range(1, N_DEV):
                make_rdma(0, d).start()

        @pl.when(i == n_blocks - 1)
        def _():
            for d in range(1, N_DEV):
                make_rdma(1, d).start()
            for h in range(2):
                for d in range(1, N_DEV):
                    make_rdma(h, d).wait()

            out_ref[...] = (
                partial_ref[0:1, :]
                + partial_ref[1:2, :]
                + jnp.sum(recv_ref[...], axis=0, keepdims=True)
            )

    return pl.pallas_call(
        body,
        grid=(n_blocks,),
        out_shape=jax.ShapeDtypeStruct((1, n), x.dtype),
        in_specs=[pl.BlockSpec((BLOCK_M, n), lambda i: (i, 0))],
        out_specs=pl.BlockSpec((1, n), lambda i: (0, 0), memory_space=pltpu.VMEM),
        scratch_shapes=[
            pltpu.VMEM((2, n), x.dtype),
            pltpu.VMEM((2 * (N_DEV - 1), n), x.dtype),
            pltpu.SemaphoreType.DMA((2 * (N_DEV - 1),)),
            pltpu.SemaphoreType.DMA((2 * (N_DEV - 1),)),
        ],
        compiler_params=pltpu.CompilerParams(
            collective_id=0,
            dimension_semantics=("arbitrary",),
        ),
    )(x)
